# initial kernel scaffold (unmeasured)
import jax
import jax.numpy as jnp
from jax import lax
from jax.experimental import pallas as pl
from jax.experimental.pallas import tpu as pltpu

N_DEV = 4


def kernel(x, Wq, Wk, Wv, Wo, t_emb, W_mod, W_ff1, W_ff2):
    B, S, D = x.shape
    R = B * S
    C = R // N_DEV
    Dh = 128
    H_loc = Wq.shape[1] // Dh
    eps = 1e-5
    scale = 0.08838834764831843

    def body(x_ref, wq_ref, wk_ref, wv_ref, wo_ref, temb_ref, wmod_ref,
             wff1_ref, wff2_ref, out_ref, xm, q, k, v, comm,
             send_sems, recv_sems, bar_sems):
        my = lax.axis_index("i")
        right = jnp.mod(my + 1, N_DEV)
        left = jnp.mod(my + 3, N_DEV)

        bsem = pltpu.get_barrier_semaphore()
        for nbr in (left, right):
            pl.semaphore_signal(bsem, inc=1, device_id=(nbr,),
                                device_id_type=pl.DeviceIdType.MESH)
        pl.semaphore_wait(bsem, 2)

        def nbar(i):
            for nbr in (left, right):
                pl.semaphore_signal(bar_sems.at[i], inc=1, device_id=(nbr,),
                                    device_id_type=pl.DeviceIdType.MESH)
            pl.semaphore_wait(bar_sems.at[i], 2)

        def all_reduce(buf, pre_bar, mid_bar):
            if pre_bar is not None:
                nbar(pre_bar)
            for s in range(N_DEV - 1):
                sc = jnp.mod(my - s + 8, N_DEV)
                rc = jnp.mod(my - s - 1 + 8, N_DEV)
                rdma = pltpu.make_async_remote_copy(
                    src_ref=buf.at[pl.ds(sc * C, C), :],
                    dst_ref=comm.at[s],
                    send_sem=send_sems.at[s],
                    recv_sem=recv_sems.at[s],
                    device_id=(right,),
                    device_id_type=pl.DeviceIdType.MESH,
                )
                rdma.start()
                rdma.wait()
                buf[pl.ds(rc * C, C), :] = buf[pl.ds(rc * C, C), :] + comm[s]
            nbar(mid_bar)
            for t in range(N_DEV - 1):
                sc = jnp.mod(my + 1 - t + 8, N_DEV)
                rdma = pltpu.make_async_remote_copy(
                    src_ref=buf.at[pl.ds(sc * C, C), :],
                    dst_ref=buf.at[pl.ds(sc * C, C), :],
                    send_sem=send_sems.at[N_DEV - 1 + t],
                    recv_sem=recv_sems.at[N_DEV - 1 + t],
                    device_id=(right,),
                    device_id_type=pl.DeviceIdType.MESH,
                )
                rdma.start()
                rdma.wait()

        def layernorm(h):
            m = jnp.mean(h, axis=1, keepdims=True)
            c = h - m
            var = jnp.mean(c * c, axis=1, keepdims=True)
            return c * lax.rsqrt(var + eps)

        mod = jnp.dot(temb_ref[:, :], wmod_ref[:, :],
                      preferred_element_type=jnp.float32)
        sa, sha, ga = mod[:, 0:D], mod[:, D:2 * D], mod[:, 2 * D:3 * D]
        sm_, shm, gm = mod[:, 3 * D:4 * D], mod[:, 4 * D:5 * D], mod[:, 5 * D:6 * D]

        for b in range(B):
            xln = layernorm(x_ref[b, :, :])
            xm[b * S:(b + 1) * S, :] = (
                xln * (1.0 + sa[b][None, :]) + sha[b][None, :])

        q[:, :] = jnp.dot(xm[:, :], wq_ref[:, :], preferred_element_type=jnp.float32)
        k[:, :] = jnp.dot(xm[:, :], wk_ref[:, :], preferred_element_type=jnp.float32)
        v[:, :] = jnp.dot(xm[:, :], wv_ref[:, :], preferred_element_type=jnp.float32)

        for b in range(B):
            r0 = b * S
            for h in range(H_loc):
                c0 = h * Dh
                qh = q[r0:r0 + S, c0:c0 + Dh]
                kh = k[r0:r0 + S, c0:c0 + Dh]
                vh = v[r0:r0 + S, c0:c0 + Dh]
                s_ = lax.dot_general(
                    qh, kh, (((1,), (1,)), ((), ())),
                    preferred_element_type=jnp.float32) * scale
                mx = jnp.max(s_, axis=1, keepdims=True)
                p = jnp.exp(s_ - mx)
                l = jnp.sum(p, axis=1, keepdims=True)
                o = jnp.dot(p, vh, preferred_element_type=jnp.float32) / l
                q[r0:r0 + S, c0:c0 + Dh] = o

        xm[:, :] = jnp.dot(q[:, :], wo_ref[:, :], preferred_element_type=jnp.float32)
        all_reduce(xm, pre_bar=None, mid_bar=0)

        for b in range(B):
            out_ref[b, :, :] = (x_ref[b, :, :]
                                + ga[b][None, :] * xm[b * S:(b + 1) * S, :])

        for b in range(B):
            xln2 = layernorm(out_ref[b, :, :])
            k[b * S:(b + 1) * S, :] = (
                xln2 * (1.0 + sm_[b][None, :]) + shm[b][None, :])
        h_ = jnp.dot(k[:, :], wff1_ref[:, :], preferred_element_type=jnp.float32)
        q[:, :] = h_ / (1.0 + jnp.exp(-h_))
        xm[:, :] = jnp.dot(q[:, :], wff2_ref[:, :], preferred_element_type=jnp.float32)
        all_reduce(xm, pre_bar=1, mid_bar=2)

        for b in range(B):
            out_ref[b, :, :] = (out_ref[b, :, :]
                                + gm[b][None, :] * xm[b * S:(b + 1) * S, :])

        nbar(3)

    return pl.pallas_call(
        body,
        out_shape=jax.ShapeDtypeStruct((B, S, D), jnp.float32),
        in_specs=[pl.BlockSpec(memory_space=pltpu.VMEM)] * 9,
        out_specs=pl.BlockSpec(memory_space=pltpu.VMEM),
        scratch_shapes=[
            pltpu.VMEM((R, D), jnp.float32),
            pltpu.VMEM((R, D), jnp.float32),
            pltpu.VMEM((R, D), jnp.float32),
            pltpu.VMEM((R, D), jnp.float32),
            pltpu.VMEM((N_DEV - 1, C, D), jnp.float32),
            pltpu.SemaphoreType.DMA((2 * (N_DEV - 1),)),
            pltpu.SemaphoreType.DMA((2 * (N_DEV - 1),)),
            pltpu.SemaphoreType.REGULAR((4,)),
        ],
        compiler_params=pltpu.CompilerParams(
            collective_id=0,
            vmem_limit_bytes=128 * 1024 * 1024,
        ),
    )(x, Wq, Wk, Wv, Wo, t_emb, W_mod, W_ff1, W_ff2)


# baseline (device time: 443081 ns/iter reference)
import jax
import jax.numpy as jnp
from jax import lax
from jax.experimental import pallas as pl
from jax.experimental.pallas import tpu as pltpu

N_DEV = 4
EPS = 1e-5
SCALE = 0.08838834764831843


def _vmem(n=1):
    return [pl.BlockSpec(memory_space=pltpu.VMEM)] * n


def _matmul(a, w):
    M, K = a.shape
    N = w.shape[1]

    def body(a_ref, w_ref, o_ref):
        o_ref[:, :] = jnp.dot(a_ref[:, :], w_ref[:, :],
                              preferred_element_type=jnp.float32)

    return pl.pallas_call(
        body,
        out_shape=jax.ShapeDtypeStruct((M, N), jnp.float32),
        in_specs=_vmem(2),
        out_specs=_vmem()[0],
    )(a, w)


def _layernorm(h):
    m = jnp.mean(h, axis=1, keepdims=True)
    c = h - m
    var = jnp.mean(c * c, axis=1, keepdims=True)
    return c * lax.rsqrt(var + EPS)


def _mod_ln(x, t_emb, W_mod, B, S, D):

    def body(x_ref, temb_ref, wmod_ref, mod_ref, xm_ref):
        mod = jnp.dot(temb_ref[:, :], wmod_ref[:, :],
                      preferred_element_type=jnp.float32)
        mod_ref[:, :] = mod
        for b in range(B):
            xln = _layernorm(x_ref[b * S:(b + 1) * S, :])
            xm_ref[b * S:(b + 1) * S, :] = (
                xln * (1.0 + mod[b, 0:D][None, :]) + mod[b, D:2 * D][None, :])

    return pl.pallas_call(
        body,
        out_shape=[
            jax.ShapeDtypeStruct((B, 6 * D), jnp.float32),
            jax.ShapeDtypeStruct((B * S, D), jnp.float32),
        ],
        in_specs=_vmem(3),
        out_specs=_vmem(2),
    )(x, t_emb, W_mod)


def _attention(q, k, v, B, S, H_loc, Dh):

    def body(q_ref, k_ref, v_ref, o_ref):
        s_ = lax.dot_general(q_ref[:, :], k_ref[:, :],
                             (((1,), (1,)), ((), ())),
                             preferred_element_type=jnp.float32) * SCALE
        mx = jnp.max(s_, axis=1, keepdims=True)
        p = jnp.exp(s_ - mx)
        l = jnp.sum(p, axis=1, keepdims=True)
        o_ref[:, :] = jnp.dot(p, v_ref[:, :],
                              preferred_element_type=jnp.float32) / l

    spec = pl.BlockSpec((S, Dh), lambda i: (i // H_loc, i % H_loc))
    return pl.pallas_call(
        body,
        grid=(B * H_loc,),
        out_shape=jax.ShapeDtypeStruct((B * S, H_loc * Dh), jnp.float32),
        in_specs=[spec, spec, spec],
        out_specs=spec,
    )(q, k, v)


def _ring_all_reduce(my, right, partial_ref, acc, comm, send_sems, recv_sems):
    R = acc.shape[0]
    C = R // N_DEV
    for s in range(N_DEV - 1):
        sc = jnp.mod(my - s + 8, N_DEV)
        rc = jnp.mod(my - s - 1 + 8, N_DEV)
        src = partial_ref if s == 0 else acc
        rdma = pltpu.make_async_remote_copy(
            src_ref=src.at[pl.ds(sc * C, C), :],
            dst_ref=comm.at[s],
            send_sem=send_sems.at[s],
            recv_sem=recv_sems.at[s],
            device_id=(right,),
            device_id_type=pl.DeviceIdType.MESH,
        )
        rdma.start()
        rdma.wait()
        acc[pl.ds(rc * C, C), :] = (
            partial_ref[pl.ds(rc * C, C), :] + comm[s])
    for t in range(N_DEV - 1):
        sc = jnp.mod(my + 1 - t + 8, N_DEV)
        rdma = pltpu.make_async_remote_copy(
            src_ref=acc.at[pl.ds(sc * C, C), :],
            dst_ref=acc.at[pl.ds(sc * C, C), :],
            send_sem=send_sems.at[N_DEV - 1 + t],
            recv_sem=recv_sems.at[N_DEV - 1 + t],
            device_id=(right,),
            device_id_type=pl.DeviceIdType.MESH,
        )
        rdma.start()
        rdma.wait()


def _barriers(bar_sems):
    my = lax.axis_index("i")
    right = jnp.mod(my + 1, N_DEV)
    left = jnp.mod(my + 3, N_DEV)

    def entry():
        bsem = pltpu.get_barrier_semaphore()
        for nbr in (left, right):
            pl.semaphore_signal(bsem, inc=1, device_id=(nbr,),
                                device_id_type=pl.DeviceIdType.MESH)
        pl.semaphore_wait(bsem, 2)

    def exit_():
        for nbr in (left, right):
            pl.semaphore_signal(bar_sems.at[0], inc=1, device_id=(nbr,),
                                device_id_type=pl.DeviceIdType.MESH)
        pl.semaphore_wait(bar_sems.at[0], 2)

    return my, right, entry, exit_


def _ar_scratch(R, D):
    C = R // N_DEV
    return [
        pltpu.VMEM((R, D), jnp.float32),
        pltpu.VMEM((N_DEV - 1, C, D), jnp.float32),
        pltpu.SemaphoreType.DMA((2 * (N_DEV - 1),)),
        pltpu.SemaphoreType.DMA((2 * (N_DEV - 1),)),
        pltpu.SemaphoreType.REGULAR((1,)),
    ]


def _ar1_residual_ln(p1, x, mod, B, S, D):

    def body(p1_ref, x_ref, mod_ref, x1_ref, xln2_ref,
             acc, comm, send_sems, recv_sems, bar_sems):
        my, right, entry, exit_ = _barriers(bar_sems)
        entry()
        _ring_all_reduce(my, right, p1_ref, acc, comm, send_sems, recv_sems)
        n_chunks = 8
        rows = B * S // n_chunks
        for c in range(n_chunks):
            b = (c * rows) // S
            r = pl.ds(c * rows, rows)
            x1 = x_ref[r, :] + mod_ref[b, 2 * D:3 * D][None, :] * acc[r, :]
            x1_ref[r, :] = x1
            xln2_ref[r, :] = (
                _layernorm(x1) * (1.0 + mod_ref[b, 3 * D:4 * D][None, :])
                + mod_ref[b, 4 * D:5 * D][None, :])
        exit_()

    return pl.pallas_call(
        body,
        out_shape=[
            jax.ShapeDtypeStruct((B * S, D), jnp.float32),
            jax.ShapeDtypeStruct((B * S, D), jnp.float32),
        ],
        in_specs=_vmem(3),
        out_specs=_vmem(2),
        scratch_shapes=_ar_scratch(B * S, D),
        compiler_params=pltpu.CompilerParams(
            collective_id=1, vmem_limit_bytes=40 * 1024 * 1024),
    )(p1, x, mod)


def _ar2_residual(p2, x1, mod, B, S, D):

    def body(p2_ref, x1_ref, mod_ref, out_ref,
             acc, comm, send_sems, recv_sems, bar_sems):
        my, right, entry, exit_ = _barriers(bar_sems)
        entry()
        _ring_all_reduce(my, right, p2_ref, acc, comm, send_sems, recv_sems)
        n_chunks = 8
        rows = B * S // n_chunks
        for c in range(n_chunks):
            b = (c * rows) // S
            r = pl.ds(c * rows, rows)
            out_ref[r, :] = (x1_ref[r, :]
                             + mod_ref[b, 5 * D:6 * D][None, :] * acc[r, :])
        exit_()

    return pl.pallas_call(
        body,
        out_shape=jax.ShapeDtypeStruct((B * S, D), jnp.float32),
        in_specs=_vmem(3),
        out_specs=_vmem()[0],
        scratch_shapes=_ar_scratch(B * S, D),
        compiler_params=pltpu.CompilerParams(
            collective_id=2, vmem_limit_bytes=40 * 1024 * 1024),
    )(p2, x1, mod)


def _ff1_silu(xln2, W_ff1):
    def body(a_ref, w_ref, o_ref):
        h = jnp.dot(a_ref[:, :], w_ref[:, :],
                    preferred_element_type=jnp.float32)
        o_ref[:, :] = h / (1.0 + jnp.exp(-h))

    return pl.pallas_call(
        body,
        out_shape=jax.ShapeDtypeStruct((xln2.shape[0], W_ff1.shape[1]),
                                       jnp.float32),
        in_specs=_vmem(2),
        out_specs=_vmem()[0],
    )(xln2, W_ff1)


def kernel(x, Wq, Wk, Wv, Wo, t_emb, W_mod, W_ff1, W_ff2):
    B, S, D = x.shape
    Dh = 128
    H_loc = Wq.shape[1] // Dh

    xf = x.reshape(B * S, D)
    mod, xm = _mod_ln(xf, t_emb, W_mod, B, S, D)
    q = _matmul(xm, Wq)
    k = _matmul(xm, Wk)
    v = _matmul(xm, Wv)
    ao = _attention(q, k, v, B, S, H_loc, Dh)
    p1 = _matmul(ao, Wo)
    x1, xln2 = _ar1_residual_ln(p1, xf, mod, B, S, D)
    h = _ff1_silu(xln2, W_ff1)
    p2 = _matmul(h, W_ff2)
    out = _ar2_residual(p2, x1, mod, B, S, D)
    return out.reshape(B, S, D)


# device time: 203321 ns/iter; 2.1792x vs baseline; 2.1792x over previous
import jax
import jax.numpy as jnp
from jax import lax
from jax.experimental import pallas as pl
from jax.experimental.pallas import tpu as pltpu

N_DEV = 4
EPS = 1e-5
SCALE = 0.08838834764831843
BF16 = jnp.bfloat16
F32 = jnp.float32


def _vmem(n=1):
    return [pl.BlockSpec(memory_space=pltpu.VMEM)] * n


def _matmul(a, w, out_dtype=BF16):
    M = a.shape[0]
    N = w.shape[1]

    def body(a_ref, w_ref, o_ref):
        o = jnp.dot(a_ref[:, :].astype(BF16), w_ref[:, :].astype(BF16),
                    preferred_element_type=F32)
        o_ref[:, :] = o.astype(out_dtype)

    return pl.pallas_call(
        body,
        out_shape=jax.ShapeDtypeStruct((M, N), out_dtype),
        in_specs=_vmem(2),
        out_specs=_vmem()[0],
    )(a, w)


def _layernorm(h):
    m = jnp.mean(h, axis=1, keepdims=True)
    c = h - m
    var = jnp.mean(c * c, axis=1, keepdims=True)
    return c * lax.rsqrt(var + EPS)


def _mod_ln(x, t_emb, W_mod, B, S, D):

    def body(x_ref, temb_ref, wmod_ref, mod_ref, xm_ref):
        mod = jnp.dot(temb_ref[:, :], wmod_ref[:, :],
                      preferred_element_type=F32)
        mod_ref[:, :] = mod
        for b in range(B):
            xln = _layernorm(x_ref[b * S:(b + 1) * S, :])
            xm_ref[b * S:(b + 1) * S, :] = (
                xln * (1.0 + mod[b, 0:D][None, :])
                + mod[b, D:2 * D][None, :]).astype(BF16)

    return pl.pallas_call(
        body,
        out_shape=[
            jax.ShapeDtypeStruct((B, 6 * D), F32),
            jax.ShapeDtypeStruct((B * S, D), BF16),
        ],
        in_specs=_vmem(3),
        out_specs=_vmem(2),
    )(x, t_emb, W_mod)


def _attention(q, k, v, B, S, H_loc, Dh):

    def body(q_ref, k_ref, v_ref, o_ref):
        s_ = lax.dot_general(q_ref[:, :], k_ref[:, :],
                             (((1,), (1,)), ((), ())),
                             preferred_element_type=F32) * SCALE
        mx = jnp.max(s_, axis=1, keepdims=True)
        p = jnp.exp(s_ - mx)
        l = jnp.sum(p, axis=1, keepdims=True)
        o = jnp.dot(p.astype(BF16), v_ref[:, :],
                    preferred_element_type=F32) / l
        o_ref[:, :] = o.astype(BF16)

    spec = pl.BlockSpec((S, Dh), lambda i: (i // H_loc, i % H_loc))
    return pl.pallas_call(
        body,
        grid=(B * H_loc,),
        out_shape=jax.ShapeDtypeStruct((B * S, H_loc * Dh), BF16),
        in_specs=[spec, spec, spec],
        out_specs=spec,
    )(q, k, v)


def _ring_all_reduce(my, partial_ref, acc, comm, send_sems, recv_sems):
    R = acc.shape[0]
    H = R // 2
    Ch = H // N_DEV
    right = jnp.mod(my + 1, N_DEV)
    left = jnp.mod(my + 3, N_DEV)

    def copy(src, s_off, dst, d_off, sem, tgt):
        return pltpu.make_async_remote_copy(
            src_ref=src.at[pl.ds(s_off, Ch), :],
            dst_ref=dst if d_off is None else dst.at[pl.ds(d_off, Ch), :],
            send_sem=send_sems.at[sem],
            recv_sem=recv_sems.at[sem],
            device_id=(tgt,),
            device_id_type=pl.DeviceIdType.MESH,
        )

    for s in range(N_DEV - 1):
        sc0 = jnp.mod(my - s + 8, N_DEV)
        rc0 = jnp.mod(my - s - 1 + 8, N_DEV)
        sc1 = jnp.mod(my + s, N_DEV)
        rc1 = jnp.mod(my + s + 1, N_DEV)
        src = partial_ref if s == 0 else acc
        r0 = copy(src, sc0 * Ch, comm.at[s], None, s, right)
        r1 = copy(src, H + sc1 * Ch, comm.at[3 + s], None, 3 + s, left)
        r0.start()
        r1.start()
        r0.wait()
        r1.wait()
        acc[pl.ds(rc0 * Ch, Ch), :] = (
            partial_ref[pl.ds(rc0 * Ch, Ch), :] + comm[s])
        acc[pl.ds(H + rc1 * Ch, Ch), :] = (
            partial_ref[pl.ds(H + rc1 * Ch, Ch), :] + comm[3 + s])
    for t in range(N_DEV - 1):
        sc0 = jnp.mod(my + 1 - t + 8, N_DEV)
        sc1 = jnp.mod(my - 1 + t + 8, N_DEV)
        r0 = copy(acc, sc0 * Ch, acc, sc0 * Ch, 6 + t, right)
        r1 = copy(acc, H + sc1 * Ch, acc, H + sc1 * Ch, 9 + t, left)
        r0.start()
        r1.start()
        r0.wait()
        r1.wait()


def _barriers(bar_sems):
    my = lax.axis_index("i")
    right = jnp.mod(my + 1, N_DEV)
    left = jnp.mod(my + 3, N_DEV)

    def entry():
        bsem = pltpu.get_barrier_semaphore()
        for nbr in (left, right):
            pl.semaphore_signal(bsem, inc=1, device_id=(nbr,),
                                device_id_type=pl.DeviceIdType.MESH)
        pl.semaphore_wait(bsem, 2)

    def exit_():
        for nbr in (left, right):
            pl.semaphore_signal(bar_sems.at[0], inc=1, device_id=(nbr,),
                                device_id_type=pl.DeviceIdType.MESH)
        pl.semaphore_wait(bar_sems.at[0], 2)

    return my, entry, exit_


def _ar_scratch(R, D):
    Ch = R // 2 // N_DEV
    return [
        pltpu.VMEM((R, D), BF16),
        pltpu.VMEM((6, Ch, D), BF16),
        pltpu.SemaphoreType.DMA((12,)),
        pltpu.SemaphoreType.DMA((12,)),
        pltpu.SemaphoreType.REGULAR((1,)),
    ]


def _ar1_residual_ln(p1, x, mod, B, S, D):

    def body(p1_ref, x_ref, mod_ref, x1_ref, xln2_ref,
             acc, comm, send_sems, recv_sems, bar_sems):
        my, entry, exit_ = _barriers(bar_sems)
        entry()
        _ring_all_reduce(my, p1_ref, acc, comm, send_sems, recv_sems)
        n_chunks = 8
        rows = B * S // n_chunks
        for c in range(n_chunks):
            b = (c * rows) // S
            r = pl.ds(c * rows, rows)
            x1 = x_ref[r, :] + mod_ref[b, 2 * D:3 * D][None, :] * acc[r, :].astype(F32)
            x1_ref[r, :] = x1
            xln2_ref[r, :] = (
                _layernorm(x1) * (1.0 + mod_ref[b, 3 * D:4 * D][None, :])
                + mod_ref[b, 4 * D:5 * D][None, :]).astype(BF16)
        exit_()

    return pl.pallas_call(
        body,
        out_shape=[
            jax.ShapeDtypeStruct((B * S, D), F32),
            jax.ShapeDtypeStruct((B * S, D), BF16),
        ],
        in_specs=_vmem(3),
        out_specs=_vmem(2),
        scratch_shapes=_ar_scratch(B * S, D),
        compiler_params=pltpu.CompilerParams(
            collective_id=1, vmem_limit_bytes=40 * 1024 * 1024),
    )(p1, x, mod)


def _ar2_residual(p2, x1, mod, B, S, D):

    def body(p2_ref, x1_ref, mod_ref, out_ref,
             acc, comm, send_sems, recv_sems, bar_sems):
        my, entry, exit_ = _barriers(bar_sems)
        entry()
        _ring_all_reduce(my, p2_ref, acc, comm, send_sems, recv_sems)
        n_chunks = 8
        rows = B * S // n_chunks
        for c in range(n_chunks):
            b = (c * rows) // S
            r = pl.ds(c * rows, rows)
            out_ref[r, :] = (x1_ref[r, :]
                             + mod_ref[b, 5 * D:6 * D][None, :]
                             * acc[r, :].astype(F32))
        exit_()

    return pl.pallas_call(
        body,
        out_shape=jax.ShapeDtypeStruct((B * S, D), F32),
        in_specs=_vmem(3),
        out_specs=_vmem()[0],
        scratch_shapes=_ar_scratch(B * S, D),
        compiler_params=pltpu.CompilerParams(
            collective_id=2, vmem_limit_bytes=40 * 1024 * 1024),
    )(p2, x1, mod)


def _ff1_silu(xln2, W_ff1):
    def body(a_ref, w_ref, o_ref):
        h = jnp.dot(a_ref[:, :], w_ref[:, :].astype(BF16),
                    preferred_element_type=F32)
        o_ref[:, :] = (h / (1.0 + jnp.exp(-h))).astype(BF16)

    return pl.pallas_call(
        body,
        out_shape=jax.ShapeDtypeStruct((xln2.shape[0], W_ff1.shape[1]), BF16),
        in_specs=_vmem(2),
        out_specs=_vmem()[0],
    )(xln2, W_ff1)


def kernel(x, Wq, Wk, Wv, Wo, t_emb, W_mod, W_ff1, W_ff2):
    B, S, D = x.shape
    Dh = 128
    H_loc = Wq.shape[1] // Dh

    xf = x.reshape(B * S, D)
    mod, xm = _mod_ln(xf, t_emb, W_mod, B, S, D)
    q = _matmul(xm, Wq)
    k = _matmul(xm, Wk)
    v = _matmul(xm, Wv)
    ao = _attention(q, k, v, B, S, H_loc, Dh)
    p1 = _matmul(ao, Wo)
    x1, xln2 = _ar1_residual_ln(p1, xf, mod, B, S, D)
    h = _ff1_silu(xln2, W_ff1)
    p2 = _matmul(h, W_ff2)
    out = _ar2_residual(p2, x1, mod, B, S, D)
    return out.reshape(B, S, D)
